# Initial kernel scaffold; baseline (speedup 1.0000x reference)
import jax
import jax.numpy as jnp
from jax.experimental import pallas as pl


def kernel(x, edge_index, W, num_iterations):
    raise NotImplementedError("write your pallas kernel here")


# speedup vs baseline: 5.4799x; 5.4799x over previous
import functools

import jax
import jax.numpy as jnp
from jax import lax
from jax.experimental import pallas as pl
from jax.experimental.pallas import tpu as pltpu
from jax.experimental.pallas import tpu_sc as plsc

N_NODES = 10000
N_EDGES = 160000
D_FEAT = 256
HALF = D_FEAT // 2

NC = 2
NS = 16
LANES = 16

PAD_N = 10240
RPT = PAD_N // NS
ZROWS = 128

E_PER_TILE = N_EDGES // NS
CHUNK = 100
N_CHUNKS = E_PER_TILE // CHUNK

_f32 = jnp.float32
_i32 = jnp.int32

GRP = 10
N_GRPS = N_CHUNKS // GRP
ZB_ROWS = 16


def _sc_mesh():
    return plsc.VectorSubcoreMesh(core_axis_name="c", subcore_axis_name="s")


def _deg_body(dst_hbm, deg_out, idx_d, ones_v, zb, deg_sp):
    c = lax.axis_index("c")
    s = lax.axis_index("s")

    @pl.when(c == 0)
    def _():
        pltpu.sync_copy(dst_hbm.at[s], idx_d)

        one16 = jnp.full((LANES,), 1.0, dtype=_f32)
        zero16 = jnp.zeros((LANES,), dtype=_f32)

        def fill(i, carry):
            for j in range(HALF // LANES):
                ones_v[i, pl.ds(j * LANES, LANES)] = one16
            return carry

        lax.fori_loop(0, CHUNK, fill, 0)

        def fillz(i, carry):
            for j in range(HALF // LANES):
                zb[i, pl.ds(j * LANES, LANES)] = zero16
            return carry

        lax.fori_loop(0, ZB_ROWS, fillz, 0)

        def zshare(k, carry):
            pltpu.sync_copy(zb, deg_sp.at[pl.ds(s * RPT + k * ZB_ROWS,
                                                ZB_ROWS)])
            return carry

        lax.fori_loop(0, RPT // ZB_ROWS, zshare, 0)
        plsc.subcore_barrier()

        def chunk_step(i, carry):
            pltpu.sync_copy(ones_v, deg_sp.at[idx_d.at[i]], add=True)
            return carry

        lax.fori_loop(0, N_CHUNKS, chunk_step, 0)
        plsc.subcore_barrier()

        pltpu.sync_copy(deg_sp.at[pl.ds(s * RPT, RPT)],
                        deg_out.at[pl.ds(s * RPT, RPT)])


_deg_kernel = functools.partial(
    pl.kernel,
    out_type=jax.ShapeDtypeStruct((PAD_N, HALF), _f32),
    mesh=_sc_mesh(),
    scratch_types=[
        pltpu.VMEM((N_CHUNKS, CHUNK), _i32),
        pltpu.VMEM((CHUNK, HALF), _f32),
        pltpu.VMEM((ZB_ROWS, HALF), _f32),
        pltpu.VMEM_SHARED((PAD_N, HALF), _f32),
    ],
)(_deg_body)


def _agg_body(h0_hbm, h1_hbm, src_hbm, dst_hbm, agg0_out, agg1_out,
              idx_s, idx_d, rows_v, zb, agg_sp, gsem, isem):
    c = lax.axis_index("c")
    s = lax.axis_index("s")

    zero16 = jnp.zeros((LANES,), dtype=_f32)

    def zrow(i, carry):
        for j in range(HALF // LANES):
            zb[i, pl.ds(j * LANES, LANES)] = zero16
        return carry

    lax.fori_loop(0, ZB_ROWS, zrow, 0)

    def zshare(k, carry):
        pltpu.sync_copy(zb, agg_sp.at[pl.ds(s * RPT + k * ZB_ROWS, ZB_ROWS)])
        return carry

    lax.fori_loop(0, RPT // ZB_ROWS, zshare, 0)
    plsc.subcore_barrier()

    def do_pass(h_hbm, out_hbm):
        def group_step(g, carry):
            cp_s = pltpu.async_copy(src_hbm.at[s, g], idx_s, isem)
            cp_d = pltpu.async_copy(dst_hbm.at[s, g], idx_d, isem)
            cp_s.wait()
            cp_d.wait()
            for j in range(GRP):
                pltpu.async_copy(h_hbm.at[idx_s.at[j]], rows_v, gsem).wait()
                pltpu.sync_copy(rows_v, agg_sp.at[idx_d.at[j]], add=True)
            return carry

        lax.fori_loop(0, N_GRPS, group_step, 0)
        plsc.subcore_barrier()
        pltpu.sync_copy(agg_sp.at[pl.ds(s * RPT, RPT)],
                        out_hbm.at[pl.ds(s * RPT, RPT)])

    @pl.when(c == 0)
    def _():
        do_pass(h0_hbm, agg0_out)

    @pl.when(c == 1)
    def _():
        do_pass(h1_hbm, agg1_out)


_agg_kernel = functools.partial(
    pl.kernel,
    out_type=(
        jax.ShapeDtypeStruct((PAD_N, HALF), _f32),
        jax.ShapeDtypeStruct((PAD_N, HALF), _f32),
    ),
    mesh=_sc_mesh(),
    scratch_types=[
        pltpu.VMEM((GRP, CHUNK), _i32),
        pltpu.VMEM((GRP, CHUNK), _i32),
        pltpu.VMEM((CHUNK, HALF), _f32),
        pltpu.VMEM((ZB_ROWS, HALF), _f32),
        pltpu.VMEM_SHARED((PAD_N, HALF), _f32),
        pltpu.SemaphoreType.DMA,
        pltpu.SemaphoreType.DMA,
    ],
)(_agg_body)


ROW_BLK = 1000


def _update_body(a0_ref, a1_ref, deg_ref, w_ref, y0_ref, y1_ref):
    inv = 1.0 / jnp.maximum(deg_ref[:, 0:1], 1.0)
    a0 = a0_ref[...] * inv
    a1 = a1_ref[...] * inv
    y = jnp.dot(a0, w_ref[0:HALF, :], preferred_element_type=_f32)
    y = y + jnp.dot(a1, w_ref[HALF:D_FEAT, :], preferred_element_type=_f32)
    y = jnp.maximum(y, 0.0)
    y0_ref[...] = y[:, 0:HALF]
    y1_ref[...] = y[:, HALF:D_FEAT]


def _tc_update(agg0, agg1, deg16, W):
    grid = (N_NODES // ROW_BLK,)
    return pl.pallas_call(
        _update_body,
        grid=grid,
        in_specs=[
            pl.BlockSpec((ROW_BLK, HALF), lambda i: (i, 0)),
            pl.BlockSpec((ROW_BLK, HALF), lambda i: (i, 0)),
            pl.BlockSpec((ROW_BLK, HALF), lambda i: (i, 0)),
            pl.BlockSpec((D_FEAT, D_FEAT), lambda i: (0, 0)),
        ],
        out_specs=[
            pl.BlockSpec((ROW_BLK, HALF), lambda i: (i, 0)),
            pl.BlockSpec((ROW_BLK, HALF), lambda i: (i, 0)),
        ],
        out_shape=[
            jax.ShapeDtypeStruct((N_NODES, HALF), _f32),
            jax.ShapeDtypeStruct((N_NODES, HALF), _f32),
        ],
    )(agg0, agg1, deg16, W)


def kernel(x, edge_index, W, num_iterations):
    src = edge_index[0].astype(_i32).reshape(NS, N_GRPS, GRP, CHUNK)
    dst = edge_index[1].astype(_i32).reshape(NS, N_GRPS, GRP, CHUNK)
    dst2 = dst.reshape(NS, N_CHUNKS, CHUNK)

    deg16 = _deg_kernel(dst2)

    def body(t, carry):
        h0, h1 = carry
        agg0, agg1 = _agg_kernel(h0, h1, src, dst)
        y0, y1 = _tc_update(agg0, agg1, deg16, W)
        return (y0, y1)

    h0, h1 = lax.fori_loop(
        0, num_iterations, body, (x[:, :HALF], x[:, HALF:])
    )
    return jnp.concatenate([h0, h1], axis=1)


# speedup vs baseline: 7.4155x; 1.3532x over previous
import functools

import jax
import jax.numpy as jnp
from jax import lax
from jax.experimental import pallas as pl
from jax.experimental.pallas import tpu as pltpu
from jax.experimental.pallas import tpu_sc as plsc

N_NODES = 10000
N_EDGES = 160000
D_FEAT = 256
HALF = D_FEAT // 2

NC = 2
NS = 16
LANES = 16

PAD_N = 10240
RPT = PAD_N // NS
ZROWS = 128

E_PER_TILE = N_EDGES // NS
CHUNK = 100
N_CHUNKS = E_PER_TILE // CHUNK

_f32 = jnp.float32
_i32 = jnp.int32

GRP = 10
N_GRPS = N_CHUNKS // GRP
ZB_ROWS = 16


def _sc_mesh():
    return plsc.VectorSubcoreMesh(core_axis_name="c", subcore_axis_name="s")


def _deg_body(dst_hbm, deg_out, idx_d, ones_v, zb, deg_sp):
    c = lax.axis_index("c")
    s = lax.axis_index("s")

    @pl.when(c == 0)
    def _():
        pltpu.sync_copy(dst_hbm.at[s], idx_d)

        one16 = jnp.full((LANES,), 1.0, dtype=_f32)
        zero16 = jnp.zeros((LANES,), dtype=_f32)

        def fill(i, carry):
            for j in range(HALF // LANES):
                ones_v[i, pl.ds(j * LANES, LANES)] = one16
            return carry

        lax.fori_loop(0, CHUNK, fill, 0)

        def fillz(i, carry):
            for j in range(HALF // LANES):
                zb[i, pl.ds(j * LANES, LANES)] = zero16
            return carry

        lax.fori_loop(0, ZB_ROWS, fillz, 0)

        def zshare(k, carry):
            pltpu.sync_copy(zb, deg_sp.at[pl.ds(s * RPT + k * ZB_ROWS,
                                                ZB_ROWS)])
            return carry

        lax.fori_loop(0, RPT // ZB_ROWS, zshare, 0)
        plsc.subcore_barrier()

        def chunk_step(i, carry):
            pltpu.sync_copy(ones_v, deg_sp.at[idx_d.at[i]], add=True)
            return carry

        lax.fori_loop(0, N_CHUNKS, chunk_step, 0)
        plsc.subcore_barrier()

        pltpu.sync_copy(deg_sp.at[pl.ds(s * RPT, RPT)],
                        deg_out.at[pl.ds(s * RPT, RPT)])


_deg_kernel = functools.partial(
    pl.kernel,
    out_type=jax.ShapeDtypeStruct((PAD_N, HALF), _f32),
    mesh=_sc_mesh(),
    scratch_types=[
        pltpu.VMEM((N_CHUNKS, CHUNK), _i32),
        pltpu.VMEM((CHUNK, HALF), _f32),
        pltpu.VMEM((ZB_ROWS, HALF), _f32),
        pltpu.VMEM_SHARED((PAD_N, HALF), _f32),
    ],
)(_deg_body)


def _agg_body(h0_hbm, h1_hbm, src_hbm, dst_hbm, agg0_out, agg1_out,
              idx_s, idx_d, rows_a, rows_b, zb, agg_sp,
              gsem_a, gsem_b, ssem_a, ssem_b, isem):
    c = lax.axis_index("c")
    s = lax.axis_index("s")

    zero16 = jnp.zeros((LANES,), dtype=_f32)

    def zrow(i, carry):
        for j in range(HALF // LANES):
            zb[i, pl.ds(j * LANES, LANES)] = zero16
        return carry

    lax.fori_loop(0, ZB_ROWS, zrow, 0)

    def zshare(k, carry):
        pltpu.sync_copy(zb, agg_sp.at[pl.ds(s * RPT + k * ZB_ROWS, ZB_ROWS)])
        return carry

    lax.fori_loop(0, RPT // ZB_ROWS, zshare, 0)
    plsc.subcore_barrier()

    rows = (rows_a, rows_b)
    gsem = (gsem_a, gsem_b)
    ssem = (ssem_a, ssem_b)

    def do_pass(h_hbm, out_hbm):
        def group_step(g, carry):
            cp_s = pltpu.async_copy(src_hbm.at[s, g], idx_s, isem)
            cp_d = pltpu.async_copy(dst_hbm.at[s, g], idx_d, isem)
            cp_s.wait()
            cp_d.wait()
            gd = [None, None]
            sd = [None, None]
            gd[0] = pltpu.async_copy(h_hbm.at[idx_s.at[0]], rows[0], gsem[0])
            for j in range(GRP):
                p = j % 2
                q = (j + 1) % 2
                if j + 1 < GRP:
                    if sd[q] is not None:
                        sd[q].wait()
                        sd[q] = None
                    gd[q] = pltpu.async_copy(h_hbm.at[idx_s.at[j + 1]],
                                             rows[q], gsem[q])
                gd[p].wait()
                sd[p] = pltpu.async_copy(rows[p], agg_sp.at[idx_d.at[j]],
                                         ssem[p], add=True)
            for p in range(2):
                if sd[p] is not None:
                    sd[p].wait()
            return carry

        lax.fori_loop(0, N_GRPS, group_step, 0)
        plsc.subcore_barrier()
        pltpu.sync_copy(agg_sp.at[pl.ds(s * RPT, RPT)],
                        out_hbm.at[pl.ds(s * RPT, RPT)])

    @pl.when(c == 0)
    def _():
        do_pass(h0_hbm, agg0_out)

    @pl.when(c == 1)
    def _():
        do_pass(h1_hbm, agg1_out)


_agg_kernel = functools.partial(
    pl.kernel,
    out_type=(
        jax.ShapeDtypeStruct((PAD_N, HALF), _f32),
        jax.ShapeDtypeStruct((PAD_N, HALF), _f32),
    ),
    mesh=_sc_mesh(),
    scratch_types=[
        pltpu.VMEM((GRP, CHUNK), _i32),
        pltpu.VMEM((GRP, CHUNK), _i32),
        pltpu.VMEM((CHUNK, HALF), _f32),
        pltpu.VMEM((CHUNK, HALF), _f32),
        pltpu.VMEM((ZB_ROWS, HALF), _f32),
        pltpu.VMEM_SHARED((PAD_N, HALF), _f32),
        pltpu.SemaphoreType.DMA,
        pltpu.SemaphoreType.DMA,
        pltpu.SemaphoreType.DMA,
        pltpu.SemaphoreType.DMA,
        pltpu.SemaphoreType.DMA,
    ],
)(_agg_body)


ROW_BLK = 1000


def _update_body(a0_ref, a1_ref, deg_ref, w_ref, y0_ref, y1_ref):
    inv = 1.0 / jnp.maximum(deg_ref[:, 0:1], 1.0)
    a0 = a0_ref[...] * inv
    a1 = a1_ref[...] * inv
    y = jnp.dot(a0, w_ref[0:HALF, :], preferred_element_type=_f32)
    y = y + jnp.dot(a1, w_ref[HALF:D_FEAT, :], preferred_element_type=_f32)
    y = jnp.maximum(y, 0.0)
    y0_ref[...] = y[:, 0:HALF]
    y1_ref[...] = y[:, HALF:D_FEAT]


def _tc_update(agg0, agg1, deg16, W):
    grid = (N_NODES // ROW_BLK,)
    return pl.pallas_call(
        _update_body,
        grid=grid,
        in_specs=[
            pl.BlockSpec((ROW_BLK, HALF), lambda i: (i, 0)),
            pl.BlockSpec((ROW_BLK, HALF), lambda i: (i, 0)),
            pl.BlockSpec((ROW_BLK, HALF), lambda i: (i, 0)),
            pl.BlockSpec((D_FEAT, D_FEAT), lambda i: (0, 0)),
        ],
        out_specs=[
            pl.BlockSpec((ROW_BLK, HALF), lambda i: (i, 0)),
            pl.BlockSpec((ROW_BLK, HALF), lambda i: (i, 0)),
        ],
        out_shape=[
            jax.ShapeDtypeStruct((N_NODES, HALF), _f32),
            jax.ShapeDtypeStruct((N_NODES, HALF), _f32),
        ],
    )(agg0, agg1, deg16, W)


def kernel(x, edge_index, W, num_iterations):
    src = edge_index[0].astype(_i32).reshape(NS, N_GRPS, GRP, CHUNK)
    dst = edge_index[1].astype(_i32).reshape(NS, N_GRPS, GRP, CHUNK)
    dst2 = dst.reshape(NS, N_CHUNKS, CHUNK)

    deg16 = _deg_kernel(dst2)

    def body(t, carry):
        h0, h1 = carry
        agg0, agg1 = _agg_kernel(h0, h1, src, dst)
        y0, y1 = _tc_update(agg0, agg1, deg16, W)
        return (y0, y1)

    h0, h1 = lax.fori_loop(
        0, num_iterations, body, (x[:, :HALF], x[:, HALF:])
    )
    return jnp.concatenate([h0, h1], axis=1)


# speedup vs baseline: 7.9076x; 1.0664x over previous
import functools

import jax
import jax.numpy as jnp
from jax import lax
from jax.experimental import pallas as pl
from jax.experimental.pallas import tpu as pltpu
from jax.experimental.pallas import tpu_sc as plsc

N_NODES = 10000
N_EDGES = 160000
D_FEAT = 256
HALF = D_FEAT // 2

NC = 2
NS = 16
LANES = 16

PAD_N = 10240
RPT = PAD_N // NS
ZROWS = 128

E_PER_TILE = N_EDGES // NS
CHUNK = 100
N_CHUNKS = E_PER_TILE // CHUNK

_f32 = jnp.float32
_i32 = jnp.int32

GRP = 10
N_GRPS = N_CHUNKS // GRP
ZB_ROWS = 16


def _sc_mesh():
    return plsc.VectorSubcoreMesh(core_axis_name="c", subcore_axis_name="s")


def _deg_body(dst_hbm, deg_out, idx_d, ones_v, zb, deg_sp, zsem, dsem):
    c = lax.axis_index("c")
    s = lax.axis_index("s")

    @pl.when(c == 0)
    def _():
        pltpu.sync_copy(dst_hbm.at[s], idx_d)

        one16 = jnp.full((LANES,), 1.0, dtype=_f32)
        zero16 = jnp.zeros((LANES,), dtype=_f32)

        def fill(i, carry):
            for j in range(HALF // LANES):
                ones_v[i, pl.ds(j * LANES, LANES)] = one16
            return carry

        lax.fori_loop(0, CHUNK, fill, 0)

        def fillz(i, carry):
            for j in range(HALF // LANES):
                zb[i, pl.ds(j * LANES, LANES)] = zero16
            return carry

        lax.fori_loop(0, ZB_ROWS, fillz, 0)

        def zshare(k, carry):
            cps = [
                pltpu.async_copy(
                    zb,
                    deg_sp.at[pl.ds(s * RPT + (8 * k + u) * ZB_ROWS,
                                    ZB_ROWS)],
                    zsem,
                )
                for u in range(8)
            ]
            for cp in cps:
                cp.wait()
            return carry

        lax.fori_loop(0, (RPT // ZB_ROWS) // 8, zshare, 0)
        plsc.subcore_barrier()

        def chunk_step(g, carry):
            cps = [
                pltpu.async_copy(
                    ones_v, deg_sp.at[idx_d.at[g * GRP + u]], dsem, add=True
                )
                for u in range(GRP)
            ]
            for cp in cps:
                cp.wait()
            return carry

        lax.fori_loop(0, N_CHUNKS // GRP, chunk_step, 0)
        plsc.subcore_barrier()

        pltpu.sync_copy(deg_sp.at[pl.ds(s * RPT, RPT)],
                        deg_out.at[pl.ds(s * RPT, RPT)])


_deg_kernel = functools.partial(
    pl.kernel,
    out_type=jax.ShapeDtypeStruct((PAD_N, HALF), _f32),
    mesh=_sc_mesh(),
    scratch_types=[
        pltpu.VMEM((N_CHUNKS, CHUNK), _i32),
        pltpu.VMEM((CHUNK, HALF), _f32),
        pltpu.VMEM((ZB_ROWS, HALF), _f32),
        pltpu.VMEM_SHARED((PAD_N, HALF), _f32),
        pltpu.SemaphoreType.DMA,
        pltpu.SemaphoreType.DMA,
    ],
)(_deg_body)


NBUF = 3


def _agg_body(h0_hbm, h1_hbm, src_hbm, dst_hbm, agg0_out, agg1_out,
              idx_s, idx_d, rows_a, rows_b, rows_c, zb, agg_sp,
              gsem_a, gsem_b, gsem_c, ssem_a, ssem_b, ssem_c, isem, zsem):
    c = lax.axis_index("c")
    s = lax.axis_index("s")

    zero16 = jnp.zeros((LANES,), dtype=_f32)

    def zrow(i, carry):
        for j in range(HALF // LANES):
            zb[i, pl.ds(j * LANES, LANES)] = zero16
        return carry

    lax.fori_loop(0, ZB_ROWS, zrow, 0)

    def zshare(k, carry):
        cps = [
            pltpu.async_copy(
                zb,
                agg_sp.at[pl.ds(s * RPT + (8 * k + u) * ZB_ROWS, ZB_ROWS)],
                zsem,
            )
            for u in range(8)
        ]
        for cp in cps:
            cp.wait()
        return carry

    lax.fori_loop(0, (RPT // ZB_ROWS) // 8, zshare, 0)
    plsc.subcore_barrier()

    rows = (rows_a, rows_b, rows_c)
    gsem = (gsem_a, gsem_b, gsem_c)
    ssem = (ssem_a, ssem_b, ssem_c)

    def do_pass(h_hbm, out_hbm):
        def group_step(g, carry):
            cp_s = pltpu.async_copy(src_hbm.at[s, g], idx_s, isem)
            cp_d = pltpu.async_copy(dst_hbm.at[s, g], idx_d, isem)
            cp_s.wait()
            cp_d.wait()
            gd = [None] * NBUF
            sd = [None] * NBUF
            gd[0] = pltpu.async_copy(h_hbm.at[idx_s.at[0]], rows[0], gsem[0])
            gd[1] = pltpu.async_copy(h_hbm.at[idx_s.at[1]], rows[1], gsem[1])
            for j in range(GRP):
                b = j % NBUF
                if j + 2 < GRP:
                    nb = (j + 2) % NBUF
                    if sd[nb] is not None:
                        sd[nb].wait()
                        sd[nb] = None
                    gd[nb] = pltpu.async_copy(h_hbm.at[idx_s.at[j + 2]],
                                              rows[nb], gsem[nb])
                gd[b].wait()
                sd[b] = pltpu.async_copy(rows[b], agg_sp.at[idx_d.at[j]],
                                         ssem[b], add=True)
            for b in range(NBUF):
                if sd[b] is not None:
                    sd[b].wait()
            return carry

        lax.fori_loop(0, N_GRPS, group_step, 0)
        plsc.subcore_barrier()
        pltpu.sync_copy(agg_sp.at[pl.ds(s * RPT, RPT)],
                        out_hbm.at[pl.ds(s * RPT, RPT)])

    @pl.when(c == 0)
    def _():
        do_pass(h0_hbm, agg0_out)

    @pl.when(c == 1)
    def _():
        do_pass(h1_hbm, agg1_out)


_agg_kernel = functools.partial(
    pl.kernel,
    out_type=(
        jax.ShapeDtypeStruct((PAD_N, HALF), _f32),
        jax.ShapeDtypeStruct((PAD_N, HALF), _f32),
    ),
    mesh=_sc_mesh(),
    scratch_types=[
        pltpu.VMEM((GRP, CHUNK), _i32),
        pltpu.VMEM((GRP, CHUNK), _i32),
        pltpu.VMEM((CHUNK, HALF), _f32),
        pltpu.VMEM((CHUNK, HALF), _f32),
        pltpu.VMEM((CHUNK, HALF), _f32),
        pltpu.VMEM((ZB_ROWS, HALF), _f32),
        pltpu.VMEM_SHARED((PAD_N, HALF), _f32),
        pltpu.SemaphoreType.DMA,
        pltpu.SemaphoreType.DMA,
        pltpu.SemaphoreType.DMA,
        pltpu.SemaphoreType.DMA,
        pltpu.SemaphoreType.DMA,
        pltpu.SemaphoreType.DMA,
        pltpu.SemaphoreType.DMA,
        pltpu.SemaphoreType.DMA,
    ],
)(_agg_body)


ROW_BLK = 1000


def _update_body(a0_ref, a1_ref, deg_ref, w_ref, y0_ref, y1_ref):
    inv = 1.0 / jnp.maximum(deg_ref[:, 0:1], 1.0)
    a0 = a0_ref[...] * inv
    a1 = a1_ref[...] * inv
    y = jnp.dot(a0, w_ref[0:HALF, :], preferred_element_type=_f32)
    y = y + jnp.dot(a1, w_ref[HALF:D_FEAT, :], preferred_element_type=_f32)
    y = jnp.maximum(y, 0.0)
    y0_ref[...] = y[:, 0:HALF]
    y1_ref[...] = y[:, HALF:D_FEAT]


def _tc_update(agg0, agg1, deg16, W):
    grid = (N_NODES // ROW_BLK,)
    return pl.pallas_call(
        _update_body,
        grid=grid,
        in_specs=[
            pl.BlockSpec((ROW_BLK, HALF), lambda i: (i, 0)),
            pl.BlockSpec((ROW_BLK, HALF), lambda i: (i, 0)),
            pl.BlockSpec((ROW_BLK, HALF), lambda i: (i, 0)),
            pl.BlockSpec((D_FEAT, D_FEAT), lambda i: (0, 0)),
        ],
        out_specs=[
            pl.BlockSpec((ROW_BLK, HALF), lambda i: (i, 0)),
            pl.BlockSpec((ROW_BLK, HALF), lambda i: (i, 0)),
        ],
        out_shape=[
            jax.ShapeDtypeStruct((N_NODES, HALF), _f32),
            jax.ShapeDtypeStruct((N_NODES, HALF), _f32),
        ],
    )(agg0, agg1, deg16, W)


def kernel(x, edge_index, W, num_iterations):
    src = edge_index[0].astype(_i32).reshape(NS, N_GRPS, GRP, CHUNK)
    dst = edge_index[1].astype(_i32).reshape(NS, N_GRPS, GRP, CHUNK)
    dst2 = dst.reshape(NS, N_CHUNKS, CHUNK)

    deg16 = _deg_kernel(dst2)

    def body(t, carry):
        h0, h1 = carry
        agg0, agg1 = _agg_kernel(h0, h1, src, dst)
        y0, y1 = _tc_update(agg0, agg1, deg16, W)
        return (y0, y1)

    h0, h1 = lax.fori_loop(
        0, num_iterations, body, (x[:, :HALF], x[:, HALF:])
    )
    return jnp.concatenate([h0, h1], axis=1)


# speedup vs baseline: 8.2818x; 1.0473x over previous
import functools

import jax
import jax.numpy as jnp
from jax import lax
from jax.experimental import pallas as pl
from jax.experimental.pallas import tpu as pltpu
from jax.experimental.pallas import tpu_sc as plsc

N_NODES = 10000
N_EDGES = 160000
D_FEAT = 256
HALF = D_FEAT // 2

NC = 2
NS = 16
LANES = 16

PAD_N = 10240
RPT = PAD_N // NS
ZROWS = 128

E_PER_TILE = N_EDGES // NS
CHUNK = 100
N_CHUNKS = E_PER_TILE // CHUNK

_f32 = jnp.float32
_i32 = jnp.int32

GRP = 10
N_GRPS = N_CHUNKS // GRP
ZB_ROWS = 16


def _sc_mesh():
    return plsc.VectorSubcoreMesh(core_axis_name="c", subcore_axis_name="s")


def _deg_body(dst_hbm, deg_out_a, deg_out_b, idx_d, ones_v, zb, deg_sp,
              zsem, dsem):
    c = lax.axis_index("c")
    s = lax.axis_index("s")

    pltpu.sync_copy(dst_hbm.at[s], idx_d)

    one16 = jnp.full((LANES,), 1.0, dtype=_f32)
    zero16 = jnp.zeros((LANES,), dtype=_f32)

    def fill(i, carry):
        for j in range(HALF // LANES):
            ones_v[i, pl.ds(j * LANES, LANES)] = one16
        return carry

    lax.fori_loop(0, CHUNK, fill, 0)

    def fillz(i, carry):
        for j in range(HALF // LANES):
            zb[i, pl.ds(j * LANES, LANES)] = zero16
        return carry

    lax.fori_loop(0, ZB_ROWS, fillz, 0)

    def zshare(k, carry):
        cps = [
            pltpu.async_copy(
                zb,
                deg_sp.at[pl.ds(s * RPT + (8 * k + u) * ZB_ROWS, ZB_ROWS)],
                zsem,
            )
            for u in range(8)
        ]
        for cp in cps:
            cp.wait()
        return carry

    lax.fori_loop(0, (RPT // ZB_ROWS) // 8, zshare, 0)
    plsc.subcore_barrier()

    half_chunks = N_CHUNKS // 2
    base = c * half_chunks

    def chunk_step(g, carry):
        cps = [
            pltpu.async_copy(
                ones_v, deg_sp.at[idx_d.at[base + g * GRP + u]], dsem,
                add=True,
            )
            for u in range(GRP)
        ]
        for cp in cps:
            cp.wait()
        return carry

    lax.fori_loop(0, half_chunks // GRP, chunk_step, 0)
    plsc.subcore_barrier()

    @pl.when(c == 0)
    def _():
        pltpu.sync_copy(deg_sp.at[pl.ds(s * RPT, RPT)],
                        deg_out_a.at[pl.ds(s * RPT, RPT)])

    @pl.when(c == 1)
    def _():
        pltpu.sync_copy(deg_sp.at[pl.ds(s * RPT, RPT)],
                        deg_out_b.at[pl.ds(s * RPT, RPT)])


_deg_kernel = functools.partial(
    pl.kernel,
    out_type=(
        jax.ShapeDtypeStruct((PAD_N, HALF), _f32),
        jax.ShapeDtypeStruct((PAD_N, HALF), _f32),
    ),
    mesh=_sc_mesh(),
    scratch_types=[
        pltpu.VMEM((N_CHUNKS, CHUNK), _i32),
        pltpu.VMEM((CHUNK, HALF), _f32),
        pltpu.VMEM((ZB_ROWS, HALF), _f32),
        pltpu.VMEM_SHARED((PAD_N, HALF), _f32),
        pltpu.SemaphoreType.DMA,
        pltpu.SemaphoreType.DMA,
    ],
)(_deg_body)


NBUF = 2


def _agg_body(h0_hbm, h1_hbm, src_hbm, dst_hbm, agg0_out, agg1_out,
              idx_s0, idx_d0, idx_s1, idx_d1, rows_a, rows_b, zb, agg_sp,
              gsem_a, gsem_b, ssem_a, ssem_b, isem0, isem1, zsem):
    c = lax.axis_index("c")
    s = lax.axis_index("s")

    zero16 = jnp.zeros((LANES,), dtype=_f32)

    def zrow(i, carry):
        for j in range(HALF // LANES):
            zb[i, pl.ds(j * LANES, LANES)] = zero16
        return carry

    lax.fori_loop(0, ZB_ROWS, zrow, 0)

    def zshare(k, carry):
        cps = [
            pltpu.async_copy(
                zb,
                agg_sp.at[pl.ds(s * RPT + (8 * k + u) * ZB_ROWS, ZB_ROWS)],
                zsem,
            )
            for u in range(8)
        ]
        for cp in cps:
            cp.wait()
        return carry

    lax.fori_loop(0, (RPT // ZB_ROWS) // 8, zshare, 0)
    plsc.subcore_barrier()

    rows = (rows_a, rows_b)
    gsem = (gsem_a, gsem_b)
    ssem = (ssem_a, ssem_b)

    def chunk_pipeline(h_hbm, idx_s, idx_d):
        gd = [None] * NBUF
        sd = [None] * NBUF
        gd[0] = pltpu.async_copy(h_hbm.at[idx_s.at[0]], rows[0], gsem[0])
        for j in range(GRP):
            p = j % 2
            q = (j + 1) % 2
            if j + 1 < GRP:
                if sd[q] is not None:
                    sd[q].wait()
                    sd[q] = None
                gd[q] = pltpu.async_copy(h_hbm.at[idx_s.at[j + 1]],
                                         rows[q], gsem[q])
            gd[p].wait()
            sd[p] = pltpu.async_copy(rows[p], agg_sp.at[idx_d.at[j]],
                                     ssem[p], add=True)
        for p in range(2):
            if sd[p] is not None:
                sd[p].wait()

    def do_pass(h_hbm, out_hbm):
        pltpu.async_copy(src_hbm.at[s, 0], idx_s0, isem0)
        pltpu.async_copy(dst_hbm.at[s, 0], idx_d0, isem0)

        def pair_step(k, carry):
            g_a = 2 * k
            g_b = g_a + 1
            pltpu.make_async_copy(src_hbm.at[s, g_a], idx_s0, isem0).wait()
            pltpu.make_async_copy(dst_hbm.at[s, g_a], idx_d0, isem0).wait()
            cp_bs = pltpu.async_copy(src_hbm.at[s, g_b], idx_s1, isem1)
            cp_bd = pltpu.async_copy(dst_hbm.at[s, g_b], idx_d1, isem1)
            chunk_pipeline(h_hbm, idx_s0, idx_d0)

            @pl.when(g_a + 2 < N_GRPS)
            def _():
                pltpu.async_copy(src_hbm.at[s, g_a + 2], idx_s0, isem0)
                pltpu.async_copy(dst_hbm.at[s, g_a + 2], idx_d0, isem0)

            cp_bs.wait()
            cp_bd.wait()
            chunk_pipeline(h_hbm, idx_s1, idx_d1)
            return carry

        lax.fori_loop(0, N_GRPS // 2, pair_step, 0)
        plsc.subcore_barrier()
        pltpu.sync_copy(agg_sp.at[pl.ds(s * RPT, RPT)],
                        out_hbm.at[pl.ds(s * RPT, RPT)])

    @pl.when(c == 0)
    def _():
        do_pass(h0_hbm, agg0_out)

    @pl.when(c == 1)
    def _():
        do_pass(h1_hbm, agg1_out)


_agg_kernel = functools.partial(
    pl.kernel,
    out_type=(
        jax.ShapeDtypeStruct((PAD_N, HALF), _f32),
        jax.ShapeDtypeStruct((PAD_N, HALF), _f32),
    ),
    mesh=_sc_mesh(),
    scratch_types=[
        pltpu.VMEM((GRP, CHUNK), _i32),
        pltpu.VMEM((GRP, CHUNK), _i32),
        pltpu.VMEM((GRP, CHUNK), _i32),
        pltpu.VMEM((GRP, CHUNK), _i32),
        pltpu.VMEM((CHUNK, HALF), _f32),
        pltpu.VMEM((CHUNK, HALF), _f32),
        pltpu.VMEM((ZB_ROWS, HALF), _f32),
        pltpu.VMEM_SHARED((PAD_N, HALF), _f32),
        pltpu.SemaphoreType.DMA,
        pltpu.SemaphoreType.DMA,
        pltpu.SemaphoreType.DMA,
        pltpu.SemaphoreType.DMA,
        pltpu.SemaphoreType.DMA,
        pltpu.SemaphoreType.DMA,
        pltpu.SemaphoreType.DMA,
    ],
)(_agg_body)


ROW_BLK = 1000


def _update_body(a0_ref, a1_ref, dega_ref, degb_ref, w_ref, y0_ref, y1_ref):
    deg = dega_ref[:, 0:1] + degb_ref[:, 0:1]
    inv = 1.0 / jnp.maximum(deg, 1.0)
    a0 = a0_ref[...] * inv
    a1 = a1_ref[...] * inv
    y = jnp.dot(a0, w_ref[0:HALF, :], preferred_element_type=_f32)
    y = y + jnp.dot(a1, w_ref[HALF:D_FEAT, :], preferred_element_type=_f32)
    y = jnp.maximum(y, 0.0)
    y0_ref[...] = y[:, 0:HALF]
    y1_ref[...] = y[:, HALF:D_FEAT]


def _tc_update(agg0, agg1, deg_a, deg_b, W):
    grid = (N_NODES // ROW_BLK,)
    return pl.pallas_call(
        _update_body,
        grid=grid,
        in_specs=[
            pl.BlockSpec((ROW_BLK, HALF), lambda i: (i, 0)),
            pl.BlockSpec((ROW_BLK, HALF), lambda i: (i, 0)),
            pl.BlockSpec((ROW_BLK, HALF), lambda i: (i, 0)),
            pl.BlockSpec((ROW_BLK, HALF), lambda i: (i, 0)),
            pl.BlockSpec((D_FEAT, D_FEAT), lambda i: (0, 0)),
        ],
        out_specs=[
            pl.BlockSpec((ROW_BLK, HALF), lambda i: (i, 0)),
            pl.BlockSpec((ROW_BLK, HALF), lambda i: (i, 0)),
        ],
        out_shape=[
            jax.ShapeDtypeStruct((N_NODES, HALF), _f32),
            jax.ShapeDtypeStruct((N_NODES, HALF), _f32),
        ],
    )(agg0, agg1, deg_a, deg_b, W)


def kernel(x, edge_index, W, num_iterations):
    src = edge_index[0].astype(_i32).reshape(NS, N_GRPS, GRP, CHUNK)
    dst = edge_index[1].astype(_i32).reshape(NS, N_GRPS, GRP, CHUNK)
    dst2 = dst.reshape(NS, N_CHUNKS, CHUNK)

    deg_a, deg_b = _deg_kernel(dst2)

    def body(t, carry):
        h0, h1 = carry
        agg0, agg1 = _agg_kernel(h0, h1, src, dst)
        y0, y1 = _tc_update(agg0, agg1, deg_a, deg_b, W)
        return (y0, y1)

    h0, h1 = lax.fori_loop(
        0, num_iterations, body, (x[:, :HALF], x[:, HALF:])
    )
    return jnp.concatenate([h0, h1], axis=1)


# speedup vs baseline: 8.7779x; 1.0599x over previous
import functools

import jax
import jax.numpy as jnp
from jax import lax
from jax.experimental import pallas as pl
from jax.experimental.pallas import tpu as pltpu
from jax.experimental.pallas import tpu_sc as plsc

N_NODES = 10000
N_EDGES = 160000
D_FEAT = 256
HALF = D_FEAT // 2

NC = 2
NS = 16
LANES = 16

PAD_N = 10240
RPT = PAD_N // NS
ZROWS = 128

E_PER_TILE = N_EDGES // NS
CHUNK = 100
N_CHUNKS = E_PER_TILE // CHUNK

_f32 = jnp.float32
_i32 = jnp.int32

GRP = 10
N_GRPS = N_CHUNKS // GRP
ZB_ROWS = 8


def _sc_mesh():
    return plsc.VectorSubcoreMesh(core_axis_name="c", subcore_axis_name="s")


def _deg_body(dst_hbm, deg_out_a, deg_out_b, idx_d, ones_v, zb, deg_sp,
              zsem, dsem):
    c = lax.axis_index("c")
    s = lax.axis_index("s")

    pltpu.sync_copy(dst_hbm.at[s], idx_d)

    one16 = jnp.full((LANES,), 1.0, dtype=_f32)
    zero16 = jnp.zeros((LANES,), dtype=_f32)

    def fill(i, carry):
        for j in range(HALF // LANES):
            ones_v[i, pl.ds(j * LANES, LANES)] = one16
        return carry

    lax.fori_loop(0, CHUNK, fill, 0)

    def fillz(i, carry):
        for j in range(HALF // LANES):
            zb[i, pl.ds(j * LANES, LANES)] = zero16
        return carry

    lax.fori_loop(0, ZB_ROWS, fillz, 0)

    def zshare(k, carry):
        cps = [
            pltpu.async_copy(
                zb,
                deg_sp.at[pl.ds(s * RPT + (8 * k + u) * ZB_ROWS, ZB_ROWS)],
                zsem,
            )
            for u in range(8)
        ]
        for cp in cps:
            cp.wait()
        return carry

    lax.fori_loop(0, (RPT // ZB_ROWS) // 8, zshare, 0)
    plsc.subcore_barrier()

    half_chunks = N_CHUNKS // 2
    base = c * half_chunks

    def chunk_step(g, carry):
        cps = [
            pltpu.async_copy(
                ones_v, deg_sp.at[idx_d.at[base + g * GRP + u]], dsem,
                add=True,
            )
            for u in range(GRP)
        ]
        for cp in cps:
            cp.wait()
        return carry

    lax.fori_loop(0, half_chunks // GRP, chunk_step, 0)
    plsc.subcore_barrier()

    @pl.when(c == 0)
    def _():
        pltpu.sync_copy(deg_sp.at[pl.ds(s * RPT, RPT)],
                        deg_out_a.at[pl.ds(s * RPT, RPT)])

    @pl.when(c == 1)
    def _():
        pltpu.sync_copy(deg_sp.at[pl.ds(s * RPT, RPT)],
                        deg_out_b.at[pl.ds(s * RPT, RPT)])


_deg_kernel = functools.partial(
    pl.kernel,
    out_type=(
        jax.ShapeDtypeStruct((PAD_N, HALF), _f32),
        jax.ShapeDtypeStruct((PAD_N, HALF), _f32),
    ),
    mesh=_sc_mesh(),
    scratch_types=[
        pltpu.VMEM((N_CHUNKS, CHUNK), _i32),
        pltpu.VMEM((CHUNK, HALF), _f32),
        pltpu.VMEM((ZB_ROWS, HALF), _f32),
        pltpu.VMEM_SHARED((PAD_N, HALF), _f32),
        pltpu.SemaphoreType.DMA,
        pltpu.SemaphoreType.DMA,
    ],
)(_deg_body)


NBUF = 3


def _agg_body(h0_hbm, h1_hbm, src_hbm, dst_hbm, agg0_out, agg1_out,
              idx_s0, idx_d0, idx_s1, idx_d1, rows_a, rows_b, rows_c,
              zb, agg_sp,
              gsem_a, gsem_b, gsem_c, ssem_a, ssem_b, ssem_c,
              isem0, isem1, zsem):
    c = lax.axis_index("c")
    s = lax.axis_index("s")

    zero16 = jnp.zeros((LANES,), dtype=_f32)

    def zrow(i, carry):
        for j in range(HALF // LANES):
            zb[i, pl.ds(j * LANES, LANES)] = zero16
        return carry

    lax.fori_loop(0, ZB_ROWS, zrow, 0)

    def zshare(k, carry):
        cps = [
            pltpu.async_copy(
                zb,
                agg_sp.at[pl.ds(s * RPT + (8 * k + u) * ZB_ROWS, ZB_ROWS)],
                zsem,
            )
            for u in range(8)
        ]
        for cp in cps:
            cp.wait()
        return carry

    lax.fori_loop(0, (RPT // ZB_ROWS) // 8, zshare, 0)
    plsc.subcore_barrier()

    rows = (rows_a, rows_b, rows_c)
    gsem = (gsem_a, gsem_b, gsem_c)
    ssem = (ssem_a, ssem_b, ssem_c)

    def chunk_pipeline(h_hbm, idx_s, idx_d):
        gd = [None] * NBUF
        sd = [None] * NBUF
        gd[0] = pltpu.async_copy(h_hbm.at[idx_s.at[0]], rows[0], gsem[0])
        gd[1] = pltpu.async_copy(h_hbm.at[idx_s.at[1]], rows[1], gsem[1])
        for j in range(GRP):
            b = j % NBUF
            if j + 2 < GRP:
                nb = (j + 2) % NBUF
                if sd[nb] is not None:
                    sd[nb].wait()
                    sd[nb] = None
                gd[nb] = pltpu.async_copy(h_hbm.at[idx_s.at[j + 2]],
                                          rows[nb], gsem[nb])
            gd[b].wait()
            sd[b] = pltpu.async_copy(rows[b], agg_sp.at[idx_d.at[j]],
                                     ssem[b], add=True)
        for b in range(NBUF):
            if sd[b] is not None:
                sd[b].wait()

    def do_pass(h_hbm, out_hbm):
        pltpu.async_copy(src_hbm.at[s, 0], idx_s0, isem0)
        pltpu.async_copy(dst_hbm.at[s, 0], idx_d0, isem0)

        def pair_step(k, carry):
            g_a = 2 * k
            g_b = g_a + 1
            pltpu.make_async_copy(src_hbm.at[s, g_a], idx_s0, isem0).wait()
            pltpu.make_async_copy(dst_hbm.at[s, g_a], idx_d0, isem0).wait()
            cp_bs = pltpu.async_copy(src_hbm.at[s, g_b], idx_s1, isem1)
            cp_bd = pltpu.async_copy(dst_hbm.at[s, g_b], idx_d1, isem1)
            chunk_pipeline(h_hbm, idx_s0, idx_d0)

            @pl.when(g_a + 2 < N_GRPS)
            def _():
                pltpu.async_copy(src_hbm.at[s, g_a + 2], idx_s0, isem0)
                pltpu.async_copy(dst_hbm.at[s, g_a + 2], idx_d0, isem0)

            cp_bs.wait()
            cp_bd.wait()
            chunk_pipeline(h_hbm, idx_s1, idx_d1)
            return carry

        lax.fori_loop(0, N_GRPS // 2, pair_step, 0)
        plsc.subcore_barrier()
        pltpu.sync_copy(agg_sp.at[pl.ds(s * RPT, RPT)],
                        out_hbm.at[pl.ds(s * RPT, RPT)])

    @pl.when(c == 0)
    def _():
        do_pass(h0_hbm, agg0_out)

    @pl.when(c == 1)
    def _():
        do_pass(h1_hbm, agg1_out)


_agg_kernel = functools.partial(
    pl.kernel,
    out_type=(
        jax.ShapeDtypeStruct((PAD_N, HALF), _f32),
        jax.ShapeDtypeStruct((PAD_N, HALF), _f32),
    ),
    mesh=_sc_mesh(),
    scratch_types=[
        pltpu.VMEM((GRP, CHUNK), _i32),
        pltpu.VMEM((GRP, CHUNK), _i32),
        pltpu.VMEM((GRP, CHUNK), _i32),
        pltpu.VMEM((GRP, CHUNK), _i32),
        pltpu.VMEM((CHUNK, HALF), _f32),
        pltpu.VMEM((CHUNK, HALF), _f32),
        pltpu.VMEM((CHUNK, HALF), _f32),
        pltpu.VMEM((ZB_ROWS, HALF), _f32),
        pltpu.VMEM_SHARED((PAD_N, HALF), _f32),
        pltpu.SemaphoreType.DMA,
        pltpu.SemaphoreType.DMA,
        pltpu.SemaphoreType.DMA,
        pltpu.SemaphoreType.DMA,
        pltpu.SemaphoreType.DMA,
        pltpu.SemaphoreType.DMA,
        pltpu.SemaphoreType.DMA,
        pltpu.SemaphoreType.DMA,
        pltpu.SemaphoreType.DMA,
    ],
)(_agg_body)


ROW_BLK = 1000


def _update_body(a0_ref, a1_ref, dega_ref, degb_ref, w_ref, y0_ref, y1_ref):
    deg = dega_ref[:, 0:1] + degb_ref[:, 0:1]
    inv = 1.0 / jnp.maximum(deg, 1.0)
    a0 = a0_ref[...] * inv
    a1 = a1_ref[...] * inv
    y = jnp.dot(a0, w_ref[0:HALF, :], preferred_element_type=_f32)
    y = y + jnp.dot(a1, w_ref[HALF:D_FEAT, :], preferred_element_type=_f32)
    y = jnp.maximum(y, 0.0)
    y0_ref[...] = y[:, 0:HALF]
    y1_ref[...] = y[:, HALF:D_FEAT]


def _tc_update(agg0, agg1, deg_a, deg_b, W):
    grid = (N_NODES // ROW_BLK,)
    return pl.pallas_call(
        _update_body,
        grid=grid,
        in_specs=[
            pl.BlockSpec((ROW_BLK, HALF), lambda i: (i, 0)),
            pl.BlockSpec((ROW_BLK, HALF), lambda i: (i, 0)),
            pl.BlockSpec((ROW_BLK, HALF), lambda i: (i, 0)),
            pl.BlockSpec((ROW_BLK, HALF), lambda i: (i, 0)),
            pl.BlockSpec((D_FEAT, D_FEAT), lambda i: (0, 0)),
        ],
        out_specs=[
            pl.BlockSpec((ROW_BLK, HALF), lambda i: (i, 0)),
            pl.BlockSpec((ROW_BLK, HALF), lambda i: (i, 0)),
        ],
        out_shape=[
            jax.ShapeDtypeStruct((N_NODES, HALF), _f32),
            jax.ShapeDtypeStruct((N_NODES, HALF), _f32),
        ],
    )(agg0, agg1, deg_a, deg_b, W)


def kernel(x, edge_index, W, num_iterations):
    src = edge_index[0].astype(_i32).reshape(NS, N_GRPS, GRP, CHUNK)
    dst = edge_index[1].astype(_i32).reshape(NS, N_GRPS, GRP, CHUNK)
    dst2 = dst.reshape(NS, N_CHUNKS, CHUNK)

    deg_a, deg_b = _deg_kernel(dst2)

    def body(t, carry):
        h0, h1 = carry
        agg0, agg1 = _agg_kernel(h0, h1, src, dst)
        y0, y1 = _tc_update(agg0, agg1, deg_a, deg_b, W)
        return (y0, y1)

    h0, h1 = lax.fori_loop(
        0, num_iterations, body, (x[:, :HALF], x[:, HALF:])
    )
    return jnp.concatenate([h0, h1], axis=1)


# speedup vs baseline: 8.8308x; 1.0060x over previous
import functools

import jax
import jax.numpy as jnp
from jax import lax
from jax.experimental import pallas as pl
from jax.experimental.pallas import tpu as pltpu
from jax.experimental.pallas import tpu_sc as plsc

N_NODES = 10000
N_EDGES = 160000
D_FEAT = 256
HALF = D_FEAT // 2

NC = 2
NS = 16
LANES = 16

PAD_N = 10240
RPT = PAD_N // NS
ZROWS = 128

E_PER_TILE = N_EDGES // NS
CHUNK = 100
N_CHUNKS = E_PER_TILE // CHUNK

_f32 = jnp.float32
_i32 = jnp.int32

GRP = 10
N_GRPS = N_CHUNKS // GRP
ZB_ROWS = 8


def _sc_mesh():
    return plsc.VectorSubcoreMesh(core_axis_name="c", subcore_axis_name="s")


def _deg_body(dst_hbm, deg_out_a, deg_out_b, idx_d, ones_v, zb, deg_sp,
              zsem, dsem):
    c = lax.axis_index("c")
    s = lax.axis_index("s")

    pltpu.sync_copy(dst_hbm.at[s], idx_d)

    one16 = jnp.full((LANES,), 1.0, dtype=_f32)
    zero16 = jnp.zeros((LANES,), dtype=_f32)

    def fill(i, carry):
        for j in range(HALF // LANES):
            ones_v[i, pl.ds(j * LANES, LANES)] = one16
        return carry

    lax.fori_loop(0, CHUNK, fill, 0)

    def fillz(i, carry):
        for j in range(HALF // LANES):
            zb[i, pl.ds(j * LANES, LANES)] = zero16
        return carry

    lax.fori_loop(0, ZB_ROWS, fillz, 0)

    def zshare(k, carry):
        cps = [
            pltpu.async_copy(
                zb,
                deg_sp.at[pl.ds(s * RPT + (8 * k + u) * ZB_ROWS, ZB_ROWS)],
                zsem,
            )
            for u in range(8)
        ]
        for cp in cps:
            cp.wait()
        return carry

    lax.fori_loop(0, (RPT // ZB_ROWS) // 8, zshare, 0)
    plsc.subcore_barrier()

    half_chunks = N_CHUNKS // 2
    base = c * half_chunks

    def chunk_step(g, carry):
        cps = [
            pltpu.async_copy(
                ones_v, deg_sp.at[idx_d.at[base + g * GRP + u]], dsem,
                add=True,
            )
            for u in range(GRP)
        ]
        for cp in cps:
            cp.wait()
        return carry

    lax.fori_loop(0, half_chunks // GRP, chunk_step, 0)
    plsc.subcore_barrier()

    @pl.when(c == 0)
    def _():
        pltpu.sync_copy(deg_sp.at[pl.ds(s * RPT, RPT)],
                        deg_out_a.at[pl.ds(s * RPT, RPT)])

    @pl.when(c == 1)
    def _():
        pltpu.sync_copy(deg_sp.at[pl.ds(s * RPT, RPT)],
                        deg_out_b.at[pl.ds(s * RPT, RPT)])


_deg_kernel = functools.partial(
    pl.kernel,
    out_type=(
        jax.ShapeDtypeStruct((PAD_N, HALF), _f32),
        jax.ShapeDtypeStruct((PAD_N, HALF), _f32),
    ),
    mesh=_sc_mesh(),
    scratch_types=[
        pltpu.VMEM((N_CHUNKS, CHUNK), _i32),
        pltpu.VMEM((CHUNK, HALF), _f32),
        pltpu.VMEM((ZB_ROWS, HALF), _f32),
        pltpu.VMEM_SHARED((PAD_N, HALF), _f32),
        pltpu.SemaphoreType.DMA,
        pltpu.SemaphoreType.DMA,
    ],
)(_deg_body)


NBUF = 3


def _agg_body(h0_hbm, h1_hbm, src_hbm, dst_hbm, agg0_out, agg1_out,
              idx_s0, idx_d0, idx_s1, idx_d1, rows_a, rows_b, rows_c,
              zb, agg_sp,
              gsem_a, gsem_b, gsem_c, ssem_a, ssem_b, ssem_c,
              isem0, isem1, zsem):
    c = lax.axis_index("c")
    s = lax.axis_index("s")

    pltpu.async_copy(src_hbm.at[s, 0], idx_s0, isem0)
    pltpu.async_copy(dst_hbm.at[s, 0], idx_d0, isem0)

    zero16 = jnp.zeros((LANES,), dtype=_f32)

    def zrow(i, carry):
        for j in range(HALF // LANES):
            zb[i, pl.ds(j * LANES, LANES)] = zero16
        return carry

    lax.fori_loop(0, ZB_ROWS, zrow, 0)

    def zshare(k, carry):
        cps = [
            pltpu.async_copy(
                zb,
                agg_sp.at[pl.ds(s * RPT + (8 * k + u) * ZB_ROWS, ZB_ROWS)],
                zsem,
            )
            for u in range(8)
        ]
        for cp in cps:
            cp.wait()
        return carry

    lax.fori_loop(0, (RPT // ZB_ROWS) // 8, zshare, 0)
    plsc.subcore_barrier()

    rows = (rows_a, rows_b, rows_c)
    gsem = (gsem_a, gsem_b, gsem_c)
    ssem = (ssem_a, ssem_b, ssem_c)

    def chunk_pipeline(h_hbm, idx_s, idx_d):
        gd = [None] * NBUF
        sd = [None] * NBUF
        gd[0] = pltpu.async_copy(h_hbm.at[idx_s.at[0]], rows[0], gsem[0])
        gd[1] = pltpu.async_copy(h_hbm.at[idx_s.at[1]], rows[1], gsem[1])
        for j in range(GRP):
            b = j % NBUF
            if j + 2 < GRP:
                nb = (j + 2) % NBUF
                if sd[nb] is not None:
                    sd[nb].wait()
                    sd[nb] = None
                gd[nb] = pltpu.async_copy(h_hbm.at[idx_s.at[j + 2]],
                                          rows[nb], gsem[nb])
            gd[b].wait()
            sd[b] = pltpu.async_copy(rows[b], agg_sp.at[idx_d.at[j]],
                                     ssem[b], add=True)
        for b in range(NBUF):
            if sd[b] is not None:
                sd[b].wait()

    def do_pass(h_hbm, out_hbm):
        def pair_step(k, carry):
            g_a = 2 * k
            g_b = g_a + 1
            pltpu.make_async_copy(src_hbm.at[s, g_a], idx_s0, isem0).wait()
            pltpu.make_async_copy(dst_hbm.at[s, g_a], idx_d0, isem0).wait()
            cp_bs = pltpu.async_copy(src_hbm.at[s, g_b], idx_s1, isem1)
            cp_bd = pltpu.async_copy(dst_hbm.at[s, g_b], idx_d1, isem1)
            chunk_pipeline(h_hbm, idx_s0, idx_d0)

            @pl.when(g_a + 2 < N_GRPS)
            def _():
                pltpu.async_copy(src_hbm.at[s, g_a + 2], idx_s0, isem0)
                pltpu.async_copy(dst_hbm.at[s, g_a + 2], idx_d0, isem0)

            cp_bs.wait()
            cp_bd.wait()
            chunk_pipeline(h_hbm, idx_s1, idx_d1)
            return carry

        lax.fori_loop(0, N_GRPS // 2, pair_step, 0)
        plsc.subcore_barrier()
        pltpu.sync_copy(agg_sp.at[pl.ds(s * RPT, RPT)],
                        out_hbm.at[pl.ds(s * RPT, RPT)])

    @pl.when(c == 0)
    def _():
        do_pass(h0_hbm, agg0_out)

    @pl.when(c == 1)
    def _():
        do_pass(h1_hbm, agg1_out)


_agg_kernel = functools.partial(
    pl.kernel,
    out_type=(
        jax.ShapeDtypeStruct((PAD_N, HALF), _f32),
        jax.ShapeDtypeStruct((PAD_N, HALF), _f32),
    ),
    mesh=_sc_mesh(),
    scratch_types=[
        pltpu.VMEM((GRP, CHUNK), _i32),
        pltpu.VMEM((GRP, CHUNK), _i32),
        pltpu.VMEM((GRP, CHUNK), _i32),
        pltpu.VMEM((GRP, CHUNK), _i32),
        pltpu.VMEM((CHUNK, HALF), _f32),
        pltpu.VMEM((CHUNK, HALF), _f32),
        pltpu.VMEM((CHUNK, HALF), _f32),
        pltpu.VMEM((ZB_ROWS, HALF), _f32),
        pltpu.VMEM_SHARED((PAD_N, HALF), _f32),
        pltpu.SemaphoreType.DMA,
        pltpu.SemaphoreType.DMA,
        pltpu.SemaphoreType.DMA,
        pltpu.SemaphoreType.DMA,
        pltpu.SemaphoreType.DMA,
        pltpu.SemaphoreType.DMA,
        pltpu.SemaphoreType.DMA,
        pltpu.SemaphoreType.DMA,
        pltpu.SemaphoreType.DMA,
    ],
)(_agg_body)


ROW_BLK = 1000


def _update_body(a0_ref, a1_ref, dega_ref, degb_ref, w_ref, y0_ref, y1_ref):
    deg = dega_ref[:, 0:1] + degb_ref[:, 0:1]
    inv = 1.0 / jnp.maximum(deg, 1.0)
    a0 = a0_ref[...] * inv
    a1 = a1_ref[...] * inv
    y = jnp.dot(a0, w_ref[0:HALF, :], preferred_element_type=_f32)
    y = y + jnp.dot(a1, w_ref[HALF:D_FEAT, :], preferred_element_type=_f32)
    y = jnp.maximum(y, 0.0)
    y0_ref[...] = y[:, 0:HALF]
    y1_ref[...] = y[:, HALF:D_FEAT]


def _tc_update(agg0, agg1, deg_a, deg_b, W):
    grid = (N_NODES // ROW_BLK,)
    return pl.pallas_call(
        _update_body,
        grid=grid,
        in_specs=[
            pl.BlockSpec((ROW_BLK, HALF), lambda i: (i, 0)),
            pl.BlockSpec((ROW_BLK, HALF), lambda i: (i, 0)),
            pl.BlockSpec((ROW_BLK, HALF), lambda i: (i, 0)),
            pl.BlockSpec((ROW_BLK, HALF), lambda i: (i, 0)),
            pl.BlockSpec((D_FEAT, D_FEAT), lambda i: (0, 0)),
        ],
        out_specs=[
            pl.BlockSpec((ROW_BLK, HALF), lambda i: (i, 0)),
            pl.BlockSpec((ROW_BLK, HALF), lambda i: (i, 0)),
        ],
        out_shape=[
            jax.ShapeDtypeStruct((N_NODES, HALF), _f32),
            jax.ShapeDtypeStruct((N_NODES, HALF), _f32),
        ],
    )(agg0, agg1, deg_a, deg_b, W)


def kernel(x, edge_index, W, num_iterations):
    src = edge_index[0].astype(_i32).reshape(NS, N_GRPS, GRP, CHUNK)
    dst = edge_index[1].astype(_i32).reshape(NS, N_GRPS, GRP, CHUNK)
    dst2 = dst.reshape(NS, N_CHUNKS, CHUNK)

    deg_a, deg_b = _deg_kernel(dst2)

    def body(t, carry):
        h0, h1 = carry
        agg0, agg1 = _agg_kernel(h0, h1, src, dst)
        y0, y1 = _tc_update(agg0, agg1, deg_a, deg_b, W)
        return (y0, y1)

    h0, h1 = lax.fori_loop(
        0, num_iterations, body, (x[:, :HALF], x[:, HALF:])
    )
    return jnp.concatenate([h0, h1], axis=1)


# speedup vs baseline: 8.8419x; 1.0013x over previous
import functools

import jax
import jax.numpy as jnp
from jax import lax
from jax.experimental import pallas as pl
from jax.experimental.pallas import tpu as pltpu
from jax.experimental.pallas import tpu_sc as plsc

N_NODES = 10000
N_EDGES = 160000
D_FEAT = 256
HALF = D_FEAT // 2

NC = 2
NS = 16
LANES = 16

PAD_N = 10240
RPT = PAD_N // NS
ZROWS = 128

E_PER_TILE = N_EDGES // NS
CHUNK = 100
N_CHUNKS = E_PER_TILE // CHUNK

_f32 = jnp.float32
_i32 = jnp.int32

GRP = 10
N_GRPS = N_CHUNKS // GRP
ZB_ROWS = 8


def _sc_mesh():
    return plsc.VectorSubcoreMesh(core_axis_name="c", subcore_axis_name="s")


def _deg_body(dst_hbm, deg_out_a, deg_out_b, idx_d, ones_v, zb, deg_sp,
              zsem, dsem):
    c = lax.axis_index("c")
    s = lax.axis_index("s")

    pltpu.sync_copy(dst_hbm.at[s], idx_d)

    one16 = jnp.full((LANES,), 1.0, dtype=_f32)
    zero16 = jnp.zeros((LANES,), dtype=_f32)

    def fill(i, carry):
        for j in range(HALF // LANES):
            ones_v[i, pl.ds(j * LANES, LANES)] = one16
        return carry

    lax.fori_loop(0, CHUNK, fill, 0)

    def fillz(i, carry):
        for j in range(HALF // LANES):
            zb[i, pl.ds(j * LANES, LANES)] = zero16
        return carry

    lax.fori_loop(0, ZB_ROWS, fillz, 0)

    def zshare(k, carry):
        cps = [
            pltpu.async_copy(
                zb,
                deg_sp.at[pl.ds(s * RPT + (8 * k + u) * ZB_ROWS, ZB_ROWS)],
                zsem,
            )
            for u in range(8)
        ]
        for cp in cps:
            cp.wait()
        return carry

    lax.fori_loop(0, (RPT // ZB_ROWS) // 8, zshare, 0)
    plsc.subcore_barrier()

    half_chunks = N_CHUNKS // 2
    base = c * half_chunks

    def chunk_step(g, carry):
        cps = [
            pltpu.async_copy(
                ones_v, deg_sp.at[idx_d.at[base + g * GRP + u]], dsem,
                add=True,
            )
            for u in range(GRP)
        ]
        for cp in cps:
            cp.wait()
        return carry

    lax.fori_loop(0, half_chunks // GRP, chunk_step, 0)
    plsc.subcore_barrier()

    @pl.when(c == 0)
    def _():
        pltpu.sync_copy(deg_sp.at[pl.ds(s * RPT, RPT)],
                        deg_out_a.at[pl.ds(s * RPT, RPT)])

    @pl.when(c == 1)
    def _():
        pltpu.sync_copy(deg_sp.at[pl.ds(s * RPT, RPT)],
                        deg_out_b.at[pl.ds(s * RPT, RPT)])


_deg_kernel = functools.partial(
    pl.kernel,
    out_type=(
        jax.ShapeDtypeStruct((PAD_N, HALF), _f32),
        jax.ShapeDtypeStruct((PAD_N, HALF), _f32),
    ),
    mesh=_sc_mesh(),
    scratch_types=[
        pltpu.VMEM((N_CHUNKS, CHUNK), _i32),
        pltpu.VMEM((CHUNK, HALF), _f32),
        pltpu.VMEM((ZB_ROWS, HALF), _f32),
        pltpu.VMEM_SHARED((PAD_N, HALF), _f32),
        pltpu.SemaphoreType.DMA,
        pltpu.SemaphoreType.DMA,
    ],
)(_deg_body)


NBUF = 3


def _agg_body(h0_hbm, h1_hbm, src_hbm, dst_hbm, agg0_out, agg1_out,
              idx_s0, idx_d0, idx_s1, idx_d1, rows_a, rows_b, rows_c,
              zb, agg_sp,
              gsem_a, gsem_b, gsem_c, ssem_a, ssem_b, ssem_c,
              isem0, isem1, zsem):
    c = lax.axis_index("c")
    s = lax.axis_index("s")

    pltpu.async_copy(src_hbm.at[s, 0], idx_s0, isem0)
    pltpu.async_copy(dst_hbm.at[s, 0], idx_d0, isem0)

    zero16 = jnp.zeros((LANES,), dtype=_f32)

    def zrow(i, carry):
        for j in range(HALF // LANES):
            zb[i, pl.ds(j * LANES, LANES)] = zero16
        return carry

    lax.fori_loop(0, ZB_ROWS, zrow, 0)

    def zshare(k, carry):
        cps = [
            pltpu.async_copy(
                zb,
                agg_sp.at[pl.ds(s * RPT + (8 * k + u) * ZB_ROWS, ZB_ROWS)],
                zsem,
            )
            for u in range(8)
        ]
        for cp in cps:
            cp.wait()
        return carry

    lax.fori_loop(0, (RPT // ZB_ROWS) // 8, zshare, 0)
    plsc.subcore_barrier()

    rows = (rows_a, rows_b, rows_c)
    gsem = (gsem_a, gsem_b, gsem_c)
    ssem = (ssem_a, ssem_b, ssem_c)

    def chunk_pipeline(h_hbm, idx_s, idx_d):
        gd = [None] * NBUF
        sd = [None] * NBUF
        gd[0] = pltpu.async_copy(h_hbm.at[idx_s.at[0]], rows[0], gsem[0])
        gd[1] = pltpu.async_copy(h_hbm.at[idx_s.at[1]], rows[1], gsem[1])
        for j in range(GRP):
            b = j % NBUF
            if j + 2 < GRP:
                nb = (j + 2) % NBUF
                if sd[nb] is not None:
                    sd[nb].wait()
                    sd[nb] = None
                gd[nb] = pltpu.async_copy(h_hbm.at[idx_s.at[j + 2]],
                                          rows[nb], gsem[nb])
            gd[b].wait()
            sd[b] = pltpu.async_copy(rows[b], agg_sp.at[idx_d.at[j]],
                                     ssem[b], add=True)
        for b in range(NBUF):
            if sd[b] is not None:
                sd[b].wait()

    def do_pass(h_hbm, out_hbm):
        def pair_step(k, carry):
            g_a = 2 * k
            g_b = g_a + 1
            pltpu.make_async_copy(src_hbm.at[s, g_a], idx_s0, isem0).wait()
            pltpu.make_async_copy(dst_hbm.at[s, g_a], idx_d0, isem0).wait()
            cp_bs = pltpu.async_copy(src_hbm.at[s, g_b], idx_s1, isem1)
            cp_bd = pltpu.async_copy(dst_hbm.at[s, g_b], idx_d1, isem1)
            chunk_pipeline(h_hbm, idx_s0, idx_d0)

            @pl.when(g_a + 2 < N_GRPS)
            def _():
                pltpu.async_copy(src_hbm.at[s, g_a + 2], idx_s0, isem0)
                pltpu.async_copy(dst_hbm.at[s, g_a + 2], idx_d0, isem0)

            cp_bs.wait()
            cp_bd.wait()
            chunk_pipeline(h_hbm, idx_s1, idx_d1)
            return carry

        lax.fori_loop(0, N_GRPS // 2, pair_step, 0)
        if N_GRPS % 2:
            g_t = N_GRPS - 1
            pltpu.make_async_copy(src_hbm.at[s, g_t], idx_s0, isem0).wait()
            pltpu.make_async_copy(dst_hbm.at[s, g_t], idx_d0, isem0).wait()
            chunk_pipeline(h_hbm, idx_s0, idx_d0)
        plsc.subcore_barrier()
        pltpu.sync_copy(agg_sp.at[pl.ds(s * RPT, RPT)],
                        out_hbm.at[pl.ds(s * RPT, RPT)])

    @pl.when(c == 0)
    def _():
        do_pass(h0_hbm, agg0_out)

    @pl.when(c == 1)
    def _():
        do_pass(h1_hbm, agg1_out)


_agg_kernel = functools.partial(
    pl.kernel,
    out_type=(
        jax.ShapeDtypeStruct((PAD_N, HALF), _f32),
        jax.ShapeDtypeStruct((PAD_N, HALF), _f32),
    ),
    mesh=_sc_mesh(),
    scratch_types=[
        pltpu.VMEM((GRP, CHUNK), _i32),
        pltpu.VMEM((GRP, CHUNK), _i32),
        pltpu.VMEM((GRP, CHUNK), _i32),
        pltpu.VMEM((GRP, CHUNK), _i32),
        pltpu.VMEM((CHUNK, HALF), _f32),
        pltpu.VMEM((CHUNK, HALF), _f32),
        pltpu.VMEM((CHUNK, HALF), _f32),
        pltpu.VMEM((ZB_ROWS, HALF), _f32),
        pltpu.VMEM_SHARED((PAD_N, HALF), _f32),
        pltpu.SemaphoreType.DMA,
        pltpu.SemaphoreType.DMA,
        pltpu.SemaphoreType.DMA,
        pltpu.SemaphoreType.DMA,
        pltpu.SemaphoreType.DMA,
        pltpu.SemaphoreType.DMA,
        pltpu.SemaphoreType.DMA,
        pltpu.SemaphoreType.DMA,
        pltpu.SemaphoreType.DMA,
    ],
)(_agg_body)


ROW_BLK = 1000


def _update_body(a0_ref, a1_ref, dega_ref, degb_ref, w_ref, y0_ref, y1_ref):
    deg = dega_ref[:, 0:1] + degb_ref[:, 0:1]
    inv = 1.0 / jnp.maximum(deg, 1.0)
    a0 = a0_ref[...] * inv
    a1 = a1_ref[...] * inv
    y = jnp.dot(a0, w_ref[0:HALF, :], preferred_element_type=_f32)
    y = y + jnp.dot(a1, w_ref[HALF:D_FEAT, :], preferred_element_type=_f32)
    y = jnp.maximum(y, 0.0)
    y0_ref[...] = y[:, 0:HALF]
    y1_ref[...] = y[:, HALF:D_FEAT]


def _tc_update(agg0, agg1, deg_a, deg_b, W):
    grid = (N_NODES // ROW_BLK,)
    return pl.pallas_call(
        _update_body,
        grid=grid,
        in_specs=[
            pl.BlockSpec((ROW_BLK, HALF), lambda i: (i, 0)),
            pl.BlockSpec((ROW_BLK, HALF), lambda i: (i, 0)),
            pl.BlockSpec((ROW_BLK, HALF), lambda i: (i, 0)),
            pl.BlockSpec((ROW_BLK, HALF), lambda i: (i, 0)),
            pl.BlockSpec((D_FEAT, D_FEAT), lambda i: (0, 0)),
        ],
        out_specs=[
            pl.BlockSpec((ROW_BLK, HALF), lambda i: (i, 0)),
            pl.BlockSpec((ROW_BLK, HALF), lambda i: (i, 0)),
        ],
        out_shape=[
            jax.ShapeDtypeStruct((N_NODES, HALF), _f32),
            jax.ShapeDtypeStruct((N_NODES, HALF), _f32),
        ],
    )(agg0, agg1, deg_a, deg_b, W)


def kernel(x, edge_index, W, num_iterations):
    src = edge_index[0].astype(_i32).reshape(NS, N_GRPS, GRP, CHUNK)
    dst = edge_index[1].astype(_i32).reshape(NS, N_GRPS, GRP, CHUNK)
    dst2 = dst.reshape(NS, N_CHUNKS, CHUNK)

    deg_a, deg_b = _deg_kernel(dst2)

    def body(t, carry):
        h0, h1 = carry
        agg0, agg1 = _agg_kernel(h0, h1, src, dst)
        y0, y1 = _tc_update(agg0, agg1, deg_a, deg_b, W)
        return (y0, y1)

    h0, h1 = lax.fori_loop(
        0, num_iterations, body, (x[:, :HALF], x[:, HALF:])
    )
    return jnp.concatenate([h0, h1], axis=1)
